# baseline (speedup 1.0000x reference)
import jax
import jax.numpy as jnp
from jax import lax
from jax.experimental import pallas as pl
from jax.experimental.pallas import tpu as pltpu
from jax.experimental.pallas import tpu_sc as plsc

_VOCAB = 1_000_000
_D = 64
_B = 4096
_S = 200
_LANES = 16

_info = plsc.get_sparse_core_info()
_NC, _NS = _info.num_cores, _info.num_subcores
_NW = _NC * _NS
_BPW = _B // _NW


def _body(idx_hbm, emb_hbm, out_hbm, idx_v, acc_v, sem):
    wid = lax.axis_index("s") * _NC + lax.axis_index("c")
    pltpu.sync_copy(idx_hbm.at[wid], idx_v)

    zeros = jnp.zeros((_LANES,), jnp.float32)

    def zero_row(i, carry):
        for k in range(_D // _LANES):
            acc_v[i, pl.ds(k * _LANES, _LANES)] = zeros
        return carry

    lax.fori_loop(0, _BPW, zero_row, 0)

    def issue(j, carry):
        pltpu.async_copy(emb_hbm.at[idx_v.at[j]], acc_v, sem, add=True)
        return carry

    lax.fori_loop(0, _S, issue, 0)

    def drain(j, carry):
        pltpu.make_async_copy(emb_hbm.at[idx_v.at[0]], acc_v, sem).wait()
        return carry

    lax.fori_loop(0, _S, drain, 0)

    scale = jnp.full((_LANES,), 1.0 / _S, jnp.float32)

    def scale_row(i, carry):
        for k in range(_D // _LANES):
            sl = pl.ds(k * _LANES, _LANES)
            acc_v[i, sl] = acc_v[i, sl] * scale
        return carry

    lax.fori_loop(0, _BPW, scale_row, 0)

    pltpu.sync_copy(acc_v, out_hbm.at[pl.ds(wid * _BPW, _BPW)])


def kernel(text_tokens, emb):
    idx3 = text_tokens.astype(jnp.int32).reshape(_NW, _BPW, _S).transpose(0, 2, 1)
    mesh = plsc.VectorSubcoreMesh(core_axis_name="c", subcore_axis_name="s")
    sc_call = pl.kernel(
        _body,
        out_type=jax.ShapeDtypeStruct((_B, _D), jnp.float32),
        mesh=mesh,
        scratch_types=[
            pltpu.VMEM((_S, _BPW), jnp.int32),
            pltpu.VMEM((_BPW, _D), jnp.float32),
            pltpu.SemaphoreType.DMA,
        ],
        compiler_params=pltpu.CompilerParams(use_tc_tiling_on_sc=False),
    )
    return sc_call(idx3, emb)
